# Initial kernel scaffold; baseline (speedup 1.0000x reference)
import jax
import jax.numpy as jnp
from jax.experimental import pallas as pl


def kernel(x):
    raise NotImplementedError("write your pallas kernel here")


# speedup vs baseline: 183.6283x; 183.6283x over previous
import jax
import jax.numpy as jnp
from jax.experimental import pallas as pl
from jax.experimental.pallas import tpu as pltpu


def _med3(a, b, c):
    return jnp.maximum(jnp.minimum(a, b), jnp.minimum(jnp.maximum(a, b), c))


def _median3x3_kernel(x_ref, o_ref):
    x = x_ref[0]
    H, W = x.shape
    zrow = jnp.zeros((1, W), x.dtype)
    xu = jnp.concatenate([zrow, x[:-1, :]], axis=0)
    xd = jnp.concatenate([x[1:, :], zrow], axis=0)

    lo = jnp.minimum(jnp.minimum(xu, x), xd)
    hi = jnp.maximum(jnp.maximum(xu, x), xd)
    mid = _med3(xu, x, xd)

    zcol = jnp.zeros((H, 1), x.dtype)

    def shl(a):
        return jnp.concatenate([zcol, a[:, :-1]], axis=1)

    def shr(a):
        return jnp.concatenate([a[:, 1:], zcol], axis=1)

    mx = jnp.maximum(jnp.maximum(shl(lo), lo), shr(lo))
    mn = jnp.minimum(jnp.minimum(shl(hi), hi), shr(hi))
    md = _med3(shl(mid), mid, shr(mid))

    o_ref[0] = _med3(mx, md, mn)


@jax.jit
def kernel(x):
    B, C, H, W = x.shape
    xf = x.reshape(B * C, H, W)
    out = pl.pallas_call(
        _median3x3_kernel,
        grid=(B * C,),
        in_specs=[pl.BlockSpec((1, H, W), lambda i: (i, 0, 0))],
        out_specs=pl.BlockSpec((1, H, W), lambda i: (i, 0, 0)),
        out_shape=jax.ShapeDtypeStruct((B * C, H, W), x.dtype),
        compiler_params=pltpu.CompilerParams(
            dimension_semantics=("parallel",),
        ),
    )(xf)
    return out.reshape(B, C, H, W)


# speedup vs baseline: 233.9662x; 1.2741x over previous
import jax
import jax.numpy as jnp
from jax.experimental import pallas as pl
from jax.experimental.pallas import tpu as pltpu


def _med3(a, b, c):
    return jnp.maximum(jnp.minimum(a, b), jnp.minimum(jnp.maximum(a, b), c))


def _median3x3_kernel(x_ref, o_ref):
    x = x_ref[0]
    H, W = x.shape

    zcol = jnp.zeros((H, 1), x.dtype)
    xl = jnp.concatenate([zcol, x[:, :-1]], axis=1)
    xr = jnp.concatenate([x[:, 1:], zcol], axis=1)

    lo = jnp.minimum(jnp.minimum(xl, x), xr)
    hi = jnp.maximum(jnp.maximum(xl, x), xr)
    mid = _med3(xl, x, xr)

    zrow = jnp.zeros((1, W), x.dtype)

    def shu(a):
        return jnp.concatenate([zrow, a[:-1, :]], axis=0)

    def shd(a):
        return jnp.concatenate([a[1:, :], zrow], axis=0)

    mx = jnp.maximum(jnp.maximum(shu(lo), lo), shd(lo))
    mn = jnp.minimum(jnp.minimum(shu(hi), hi), shd(hi))
    md = _med3(shu(mid), mid, shd(mid))

    o_ref[0] = _med3(mx, md, mn)


@jax.jit
def kernel(x):
    B, C, H, W = x.shape
    xf = x.reshape(B * C, H, W)
    out = pl.pallas_call(
        _median3x3_kernel,
        grid=(B * C,),
        in_specs=[pl.BlockSpec((1, H, W), lambda i: (i, 0, 0))],
        out_specs=pl.BlockSpec((1, H, W), lambda i: (i, 0, 0)),
        out_shape=jax.ShapeDtypeStruct((B * C, H, W), x.dtype),
        compiler_params=pltpu.CompilerParams(
            dimension_semantics=("parallel",),
        ),
    )(xf)
    return out.reshape(B, C, H, W)


# speedup vs baseline: 240.7733x; 1.0291x over previous
import jax
import jax.numpy as jnp
from jax.experimental import pallas as pl
from jax.experimental.pallas import tpu as pltpu

_P = 2


def _med3(a, b, c):
    return jnp.maximum(jnp.minimum(a, b), jnp.minimum(jnp.maximum(a, b), c))


def _median3x3_kernel(x_ref, o_ref):
    x = x_ref[...]
    P, H, W = x.shape

    zcol = jnp.zeros((P, H, 1), x.dtype)
    xl = jnp.concatenate([zcol, x[:, :, :-1]], axis=2)
    xr = jnp.concatenate([x[:, :, 1:], zcol], axis=2)

    mnh = jnp.minimum(x, xr)
    mxh = jnp.maximum(x, xr)
    lo = jnp.minimum(xl, mnh)
    hi = jnp.maximum(xl, mxh)
    mid = jnp.maximum(jnp.minimum(xl, mxh), mnh)

    zrow = jnp.zeros((P, 1, W), x.dtype)

    def shu(a):
        return jnp.concatenate([zrow, a[:, :-1, :]], axis=1)

    def shd(a):
        return jnp.concatenate([a[:, 1:, :], zrow], axis=1)

    mx = jnp.maximum(jnp.maximum(shu(lo), lo), shd(lo))
    mn = jnp.minimum(jnp.minimum(shu(hi), hi), shd(hi))
    md = _med3(shu(mid), mid, shd(mid))

    o_ref[...] = _med3(mx, md, mn)


@jax.jit
def kernel(x):
    B, C, H, W = x.shape
    N = B * C
    xf = x.reshape(N, H, W)
    out = pl.pallas_call(
        _median3x3_kernel,
        grid=(N // _P,),
        in_specs=[pl.BlockSpec((_P, H, W), lambda i: (i, 0, 0))],
        out_specs=pl.BlockSpec((_P, H, W), lambda i: (i, 0, 0)),
        out_shape=jax.ShapeDtypeStruct((N, H, W), x.dtype),
        compiler_params=pltpu.CompilerParams(
            dimension_semantics=("parallel",),
        ),
    )(xf)
    return out.reshape(B, C, H, W)
